# baseline (speedup 1.0000x reference)
import functools
import jax
import jax.numpy as jnp
from jax import lax
from jax.experimental import pallas as pl
from jax.experimental.pallas import tpu as pltpu
from jax.experimental.pallas import tpu_sc as plsc

_N = 10000
_E = 320000
_IN = 128
_HID = 64
_OUT = 128

_NC = 2
_NS = 16
_NW = _NC * _NS
_EPW = _E // _NW
_CH = 80
_NCHUNK = _EPW // _CH
_DCH = 80
_DNC = _EPW // _DCH

_ND = 10240
_DPT = _ND // _NS

_CT = 10
_RPT = _N // _CT

_mesh = plsc.VectorSubcoreMesh(core_axis_name="c", subcore_axis_name="s")


@functools.partial(
    pl.kernel,
    out_type=jax.ShapeDtypeStruct((_NC * _ND,), jnp.float32),
    mesh=_mesh,
    scratch_types=[
        pltpu.VMEM_SHARED((_ND,), jnp.float32),
        pltpu.VMEM((_DNC, _DCH), jnp.int32),
        pltpu.VMEM((_DCH,), jnp.float32),
        pltpu.VMEM((_DPT,), jnp.float32),
        pltpu.SemaphoreType.DMA,
        pltpu.SemaphoreType.DMA,
    ],
)
def _deg_kernel(dst_hbm, out_hbm, acc, didx, ones_v, zb, isem, dsem):
    c = lax.axis_index("c")
    s = lax.axis_index("s")
    wid = c * _NS + s

    icopy = pltpu.async_copy(dst_hbm.at[wid], didx, isem)

    def fill(i, _):
        zb[pl.ds(i * 16, 16)] = jnp.zeros((16,), jnp.float32)
        return 0
    lax.fori_loop(0, _DPT // 16, fill, 0)
    for k in range(_DCH // 16):
        ones_v[pl.ds(k * 16, 16)] = jnp.ones((16,), jnp.float32)

    pltpu.sync_copy(zb, acc.at[pl.ds(s * _DPT, _DPT)])
    plsc.subcore_barrier()
    icopy.wait()

    def dscat(j):
        pltpu.async_copy(ones_v, acc.at[didx.at[j]], dsem, add=True)

    def dwait(j):
        pltpu.make_async_copy(ones_v, acc.at[didx.at[j]], dsem).wait()

    for b in range(5):
        dscat(b)

    def dgroup(g, _):
        for b in range(5):
            dscat((g + 1) * 5 + b)
        for b in range(5):
            dwait(g * 5 + b)
        return 0
    lax.fori_loop(0, _DNC // 5 - 1, dgroup, 0)
    for b in range(5):
        dwait(_DNC - 5 + b)

    plsc.subcore_barrier()
    pltpu.sync_copy(acc.at[pl.ds(s * _DPT, _DPT)],
                    out_hbm.at[pl.ds(c * _ND + s * _DPT, _DPT)])


def _make_agg(D):
    @functools.partial(
        pl.kernel,
        out_type=jax.ShapeDtypeStruct((_NC, _N, D), jnp.float32),
        mesh=_mesh,
        compiler_params=pltpu.CompilerParams(use_tc_tiling_on_sc=False),
        scratch_types=[
            pltpu.VMEM_SHARED((_N, D), jnp.float32),
            pltpu.VMEM((_EPW,), jnp.int32),
            pltpu.VMEM((_EPW,), jnp.int32),
            pltpu.VMEM((_CH, D), jnp.float32),
            pltpu.VMEM((_CH, D), jnp.float32),
            pltpu.VMEM((_CH, D), jnp.float32),
            pltpu.VMEM((_CH, D), jnp.float32),
            pltpu.SemaphoreType.DMA,
            pltpu.SemaphoreType.DMA,
            pltpu.SemaphoreType.DMA,
            pltpu.SemaphoreType.DMA,
            pltpu.SemaphoreType.DMA,
            pltpu.SemaphoreType.DMA,
            pltpu.SemaphoreType.DMA,
            pltpu.SemaphoreType.DMA,
            pltpu.SemaphoreType.DMA,
        ],
    )
    def _agg(hp_hbm, src_hbm, dst_hbm, out_hbm, acc, sidx, dall,
             r0, r1, r2, r3,
             isem, g0, g1, g2, g3, s0, s1, s2, s3):
        rows = [r0, r1, r2, r3]
        gs = [g0, g1, g2, g3]
        ss = [s0, s1, s2, s3]
        c = lax.axis_index("c")
        s = lax.axis_index("s")
        wid = c * _NS + s
        ebase = wid * _EPW

        ic1 = pltpu.async_copy(src_hbm.at[pl.ds(ebase, _EPW)], sidx, isem)
        ic2 = pltpu.async_copy(dst_hbm.at[pl.ds(ebase, _EPW)], dall, isem)

        def zrow(r, _):
            for j in range(D // 16):
                r0[r, pl.ds(j * 16, 16)] = jnp.zeros((16,), jnp.float32)
            return 0
        lax.fori_loop(0, _CH, zrow, 0)

        @pl.when(s < _CT)
        def _zero():
            zc = 40
            nz = _RPT // zc
            for t0 in range(0, nz, 5):
                for t in range(t0, min(t0 + 5, nz)):
                    pltpu.async_copy(
                        r0.at[pl.ds(0, zc)],
                        acc.at[pl.ds(s * _RPT + t * zc, zc)], g0)
                for t in range(t0, min(t0 + 5, nz)):
                    pltpu.make_async_copy(
                        r0.at[pl.ds(0, zc)],
                        acc.at[pl.ds(s * _RPT + t * zc, zc)], g0).wait()

        plsc.subcore_barrier()
        ic1.wait()
        ic2.wait()

        def gather(j, b):
            pltpu.async_copy(hp_hbm.at[sidx.at[pl.ds(j * _CH, _CH)]],
                             rows[b], gs[b])

        def wait_gather(j, b):
            pltpu.make_async_copy(hp_hbm.at[sidx.at[pl.ds(j * _CH, _CH)]],
                                  rows[b], gs[b]).wait()

        def scatter(j, b):
            pltpu.async_copy(rows[b], acc.at[dall.at[pl.ds(j * _CH, _CH)]],
                             ss[b], add=True)

        def wait_scatter(j, b):
            pltpu.make_async_copy(rows[b],
                                  acc.at[dall.at[pl.ds(j * _CH, _CH)]],
                                  ss[b]).wait()

        gather(0, 0)
        gather(1, 1)
        for j in (0, 1):
            gather(j + 2, j + 2)
            wait_gather(j, j)
            scatter(j, j)

        def group(g, _):
            for b in range(4):
                j = 2 + g * 4 + b
                jb = (2 + b) % 4
                wait_scatter(j - 2, b)
                gather(j + 2, b)
                wait_gather(j, jb)
                scatter(j, jb)
            return 0
        lax.fori_loop(0, (_NCHUNK - 5) // 4, group, 0)

        jt = _NCHUNK - 3
        wait_scatter(jt - 2, 0)
        gather(jt + 2, 0)
        wait_gather(jt, 2)
        scatter(jt, 2)
        wait_gather(jt + 1, 3)
        scatter(jt + 1, 3)
        wait_gather(jt + 2, 0)
        scatter(jt + 2, 0)
        wait_scatter(jt - 1, 1)
        wait_scatter(jt, 2)
        wait_scatter(jt + 1, 3)
        wait_scatter(jt + 2, 0)

        plsc.subcore_barrier()

        @pl.when(s < _CT)
        def _copy_out():
            pltpu.sync_copy(acc.at[pl.ds(s * _RPT, _RPT)],
                            out_hbm.at[c, pl.ds(s * _RPT, _RPT)])
    return _agg


_agg64 = _make_agg(_HID)


_GB = 1000
_GRID = _N // _GB


def _mm1_body(x_ref, w_ref, d0_ref, d1_ref, o_ref):
    dinv = lax.rsqrt(d0_ref[...] + d1_ref[...] + 1.0)
    o_ref[...] = jnp.dot(x_ref[...], w_ref[...],
                         preferred_element_type=jnp.float32) * dinv


def _mm1(x, W1, deg0, deg1):
    return pl.pallas_call(
        _mm1_body,
        grid=(_GRID,),
        in_specs=[
            pl.BlockSpec((_GB, _IN), lambda i: (i, 0)),
            pl.BlockSpec((_IN, _HID), lambda i: (0, 0)),
            pl.BlockSpec((_GB, 1), lambda i: (i, 0)),
            pl.BlockSpec((_GB, 1), lambda i: (i, 0)),
        ],
        out_specs=pl.BlockSpec((_GB, _HID), lambda i: (i, 0)),
        out_shape=jax.ShapeDtypeStruct((_N, _HID), jnp.float32),
    )(x, W1, deg0, deg1)


def _mid_body(a0_ref, a1_ref, hp_ref, d0_ref, d1_ref, b1_ref, o_ref):
    dinv = lax.rsqrt(d0_ref[...] + d1_ref[...] + 1.0)
    acc = a0_ref[...] + a1_ref[...] + hp_ref[...]
    o_ref[...] = jnp.maximum(acc * dinv + b1_ref[...], 0.0) * dinv


def _mid(a0, a1, hp, deg0, deg1, b1):
    return pl.pallas_call(
        _mid_body,
        grid=(_GRID,),
        in_specs=[
            pl.BlockSpec((_GB, _HID), lambda i: (i, 0)),
            pl.BlockSpec((_GB, _HID), lambda i: (i, 0)),
            pl.BlockSpec((_GB, _HID), lambda i: (i, 0)),
            pl.BlockSpec((_GB, 1), lambda i: (i, 0)),
            pl.BlockSpec((_GB, 1), lambda i: (i, 0)),
            pl.BlockSpec((1, _HID), lambda i: (0, 0)),
        ],
        out_specs=pl.BlockSpec((_GB, _HID), lambda i: (i, 0)),
        out_shape=jax.ShapeDtypeStruct((_N, _HID), jnp.float32),
    )(a0, a1, hp, deg0, deg1, b1)


def _fin_body(g0_ref, g1_ref, hp_ref, d0_ref, d1_ref, b2_ref, w2_ref, o_ref):
    dinv = lax.rsqrt(d0_ref[...] + d1_ref[...] + 1.0)
    t = (g0_ref[...] + g1_ref[...] + hp_ref[...]) * dinv
    o_ref[...] = jnp.dot(t, w2_ref[...], precision=lax.Precision.HIGHEST,
                         preferred_element_type=jnp.float32) + b2_ref[...]


def _fin(g0, g1, hp, deg0, deg1, b2, W2):
    return pl.pallas_call(
        _fin_body,
        grid=(_GRID,),
        in_specs=[
            pl.BlockSpec((_GB, _HID), lambda i: (i, 0)),
            pl.BlockSpec((_GB, _HID), lambda i: (i, 0)),
            pl.BlockSpec((_GB, _HID), lambda i: (i, 0)),
            pl.BlockSpec((_GB, 1), lambda i: (i, 0)),
            pl.BlockSpec((_GB, 1), lambda i: (i, 0)),
            pl.BlockSpec((1, _OUT), lambda i: (0, 0)),
            pl.BlockSpec((_HID, _OUT), lambda i: (0, 0)),
        ],
        out_specs=pl.BlockSpec((_GB, _OUT), lambda i: (i, 0)),
        out_shape=jax.ShapeDtypeStruct((_N, _OUT), jnp.float32),
    )(g0, g1, hp, deg0, deg1, b2, W2)


def kernel(x, edge_index, W1, b1, W2, b2):
    src = edge_index[0]
    dst = edge_index[1]
    dstd = edge_index[1].reshape(_NW, _DNC, _DCH)

    degp = _deg_kernel(dstd).reshape(_NC, _ND)
    deg0 = degp[0, :_N, None]
    deg1 = degp[1, :_N, None]

    h1p = _mm1(x, W1, deg0, deg1)
    agg1 = _agg64(h1p, src, dst)
    h2p = _mid(agg1[0], agg1[1], h1p, deg0, deg1,
               b1.reshape(1, _HID))
    agg2 = _agg64(h2p, src, dst)
    return _fin(agg2[0], agg2[1], h2p, deg0, deg1,
                b2.reshape(1, _OUT), W2)


# speedup vs baseline: 1.0051x; 1.0051x over previous
import functools
import jax
import jax.numpy as jnp
from jax import lax
from jax.experimental import pallas as pl
from jax.experimental.pallas import tpu as pltpu
from jax.experimental.pallas import tpu_sc as plsc

_N = 10000
_E = 320000
_IN = 128
_HID = 64
_OUT = 128

_NC = 2
_NS = 16
_NW = _NC * _NS
_EPW = _E // _NW
_CH = 128
_NF = _EPW // _CH
_REM = _EPW - _NF * _CH
_RBASE = _NF * _CH
_DCH = 80
_DNC = _EPW // _DCH

_ND = 10240
_DPT = _ND // _NS

_CT = 10
_RPT = _N // _CT

_mesh = plsc.VectorSubcoreMesh(core_axis_name="c", subcore_axis_name="s")


@functools.partial(
    pl.kernel,
    out_type=jax.ShapeDtypeStruct((_NC * _ND,), jnp.float32),
    mesh=_mesh,
    scratch_types=[
        pltpu.VMEM_SHARED((_ND,), jnp.float32),
        pltpu.VMEM((_DNC, _DCH), jnp.int32),
        pltpu.VMEM((_DCH,), jnp.float32),
        pltpu.VMEM((_DPT,), jnp.float32),
        pltpu.SemaphoreType.DMA,
        pltpu.SemaphoreType.DMA,
    ],
)
def _deg_kernel(dst_hbm, out_hbm, acc, didx, ones_v, zb, isem, dsem):
    c = lax.axis_index("c")
    s = lax.axis_index("s")
    wid = c * _NS + s

    icopy = pltpu.async_copy(dst_hbm.at[wid], didx, isem)

    def fill(i, _):
        zb[pl.ds(i * 16, 16)] = jnp.zeros((16,), jnp.float32)
        return 0
    lax.fori_loop(0, _DPT // 16, fill, 0)
    for k in range(_DCH // 16):
        ones_v[pl.ds(k * 16, 16)] = jnp.ones((16,), jnp.float32)

    pltpu.sync_copy(zb, acc.at[pl.ds(s * _DPT, _DPT)])
    plsc.subcore_barrier()
    icopy.wait()

    def dscat(j):
        pltpu.async_copy(ones_v, acc.at[didx.at[j]], dsem, add=True)

    def dwait(j):
        pltpu.make_async_copy(ones_v, acc.at[didx.at[j]], dsem).wait()

    for b in range(5):
        dscat(b)

    def dgroup(g, _):
        for b in range(5):
            dscat((g + 1) * 5 + b)
        for b in range(5):
            dwait(g * 5 + b)
        return 0
    lax.fori_loop(0, _DNC // 5 - 1, dgroup, 0)
    for b in range(5):
        dwait(_DNC - 5 + b)

    plsc.subcore_barrier()
    pltpu.sync_copy(acc.at[pl.ds(s * _DPT, _DPT)],
                    out_hbm.at[pl.ds(c * _ND + s * _DPT, _DPT)])


def _make_agg(D):
    @functools.partial(
        pl.kernel,
        out_type=jax.ShapeDtypeStruct((_NC, _N, D), jnp.float32),
        mesh=_mesh,
        compiler_params=pltpu.CompilerParams(use_tc_tiling_on_sc=False),
        scratch_types=[
            pltpu.VMEM_SHARED((_N, D), jnp.float32),
            pltpu.VMEM((_EPW,), jnp.int32),
            pltpu.VMEM((_EPW,), jnp.int32),
            pltpu.VMEM((_CH, D), jnp.float32),
            pltpu.VMEM((_CH, D), jnp.float32),
            pltpu.VMEM((_CH, D), jnp.float32),
            pltpu.VMEM((_CH, D), jnp.float32),
            pltpu.SemaphoreType.DMA,
            pltpu.SemaphoreType.DMA,
            pltpu.SemaphoreType.DMA,
            pltpu.SemaphoreType.DMA,
            pltpu.SemaphoreType.DMA,
            pltpu.SemaphoreType.DMA,
            pltpu.SemaphoreType.DMA,
            pltpu.SemaphoreType.DMA,
            pltpu.SemaphoreType.DMA,
        ],
    )
    def _agg(hp_hbm, src_hbm, dst_hbm, out_hbm, acc, sidx, dall,
             r0, r1, r2, r3,
             isem, g0, g1, g2, g3, s0, s1, s2, s3):
        rows = [r0, r1, r2, r3]
        gs = [g0, g1, g2, g3]
        ss = [s0, s1, s2, s3]
        c = lax.axis_index("c")
        s = lax.axis_index("s")
        wid = c * _NS + s
        ebase = wid * _EPW

        ic1 = pltpu.async_copy(src_hbm.at[pl.ds(ebase, _EPW)], sidx, isem)
        ic2 = pltpu.async_copy(dst_hbm.at[pl.ds(ebase, _EPW)], dall, isem)

        def zrow(r, _):
            for j in range(D // 16):
                r0[r, pl.ds(j * 16, 16)] = jnp.zeros((16,), jnp.float32)
            return 0
        lax.fori_loop(0, 40, zrow, 0)

        @pl.when(s < _CT)
        def _zero():
            zc = 40
            nz = _RPT // zc
            for t0 in range(0, nz, 5):
                for t in range(t0, min(t0 + 5, nz)):
                    pltpu.async_copy(
                        r0.at[pl.ds(0, zc)],
                        acc.at[pl.ds(s * _RPT + t * zc, zc)], g0)
                for t in range(t0, min(t0 + 5, nz)):
                    pltpu.make_async_copy(
                        r0.at[pl.ds(0, zc)],
                        acc.at[pl.ds(s * _RPT + t * zc, zc)], g0).wait()

        plsc.subcore_barrier()
        ic1.wait()
        ic2.wait()

        def gather(j, b):
            pltpu.async_copy(hp_hbm.at[sidx.at[pl.ds(j * _CH, _CH)]],
                             rows[b], gs[b])

        def wait_gather(j, b):
            pltpu.make_async_copy(hp_hbm.at[sidx.at[pl.ds(j * _CH, _CH)]],
                                  rows[b], gs[b]).wait()

        def scatter(j, b):
            pltpu.async_copy(rows[b], acc.at[dall.at[pl.ds(j * _CH, _CH)]],
                             ss[b], add=True)

        def wait_scatter(j, b):
            pltpu.make_async_copy(rows[b],
                                  acc.at[dall.at[pl.ds(j * _CH, _CH)]],
                                  ss[b]).wait()

        gather(0, 0)
        gather(1, 1)
        for j in (0, 1):
            gather(j + 2, j + 2)
            wait_gather(j, j)
            scatter(j, j)

        def group(g, _):
            for b in range(4):
                j = 2 + g * 4 + b
                jb = (2 + b) % 4
                wait_scatter(j - 2, b)
                gather(j + 2, b)
                wait_gather(j, jb)
                scatter(j, jb)
            return 0
        lax.fori_loop(0, (_NF - 6) // 4, group, 0)

        jt = _NF - 4
        wait_scatter(jt - 2, 0)
        gather(jt + 2, 0)
        wait_gather(jt, 2)
        scatter(jt, 2)
        wait_scatter(jt - 1, 1)
        gather(jt + 3, 1)
        wait_gather(jt + 1, 3)
        scatter(jt + 1, 3)
        wait_gather(jt + 2, 0)
        scatter(jt + 2, 0)
        wait_gather(jt + 3, 1)
        scatter(jt + 3, 1)
        wait_scatter(jt, 2)
        wait_scatter(jt + 1, 3)
        wait_scatter(jt + 2, 0)
        wait_scatter(jt + 3, 1)

        rsl = r2.at[pl.ds(0, _REM)]
        pltpu.async_copy(hp_hbm.at[sidx.at[pl.ds(_RBASE, _REM)]], rsl, g2)
        pltpu.make_async_copy(
            hp_hbm.at[sidx.at[pl.ds(_RBASE, _REM)]], rsl, g2).wait()
        pltpu.async_copy(rsl, acc.at[dall.at[pl.ds(_RBASE, _REM)]],
                         s2, add=True)
        pltpu.make_async_copy(rsl, acc.at[dall.at[pl.ds(_RBASE, _REM)]],
                              s2).wait()

        plsc.subcore_barrier()

        @pl.when(s < _CT)
        def _copy_out():
            pltpu.sync_copy(acc.at[pl.ds(s * _RPT, _RPT)],
                            out_hbm.at[c, pl.ds(s * _RPT, _RPT)])
    return _agg


_agg64 = _make_agg(_HID)


_GB = 1000
_GRID = _N // _GB


def _mm1_body(x_ref, w_ref, d0_ref, d1_ref, o_ref):
    dinv = lax.rsqrt(d0_ref[...] + d1_ref[...] + 1.0)
    o_ref[...] = jnp.dot(x_ref[...], w_ref[...],
                         preferred_element_type=jnp.float32) * dinv


def _mm1(x, W1, deg0, deg1):
    return pl.pallas_call(
        _mm1_body,
        grid=(_GRID,),
        in_specs=[
            pl.BlockSpec((_GB, _IN), lambda i: (i, 0)),
            pl.BlockSpec((_IN, _HID), lambda i: (0, 0)),
            pl.BlockSpec((_GB, 1), lambda i: (i, 0)),
            pl.BlockSpec((_GB, 1), lambda i: (i, 0)),
        ],
        out_specs=pl.BlockSpec((_GB, _HID), lambda i: (i, 0)),
        out_shape=jax.ShapeDtypeStruct((_N, _HID), jnp.float32),
    )(x, W1, deg0, deg1)


def _mid_body(a0_ref, a1_ref, hp_ref, d0_ref, d1_ref, b1_ref, o_ref):
    dinv = lax.rsqrt(d0_ref[...] + d1_ref[...] + 1.0)
    acc = a0_ref[...] + a1_ref[...] + hp_ref[...]
    o_ref[...] = jnp.maximum(acc * dinv + b1_ref[...], 0.0) * dinv


def _mid(a0, a1, hp, deg0, deg1, b1):
    return pl.pallas_call(
        _mid_body,
        grid=(_GRID,),
        in_specs=[
            pl.BlockSpec((_GB, _HID), lambda i: (i, 0)),
            pl.BlockSpec((_GB, _HID), lambda i: (i, 0)),
            pl.BlockSpec((_GB, _HID), lambda i: (i, 0)),
            pl.BlockSpec((_GB, 1), lambda i: (i, 0)),
            pl.BlockSpec((_GB, 1), lambda i: (i, 0)),
            pl.BlockSpec((1, _HID), lambda i: (0, 0)),
        ],
        out_specs=pl.BlockSpec((_GB, _HID), lambda i: (i, 0)),
        out_shape=jax.ShapeDtypeStruct((_N, _HID), jnp.float32),
    )(a0, a1, hp, deg0, deg1, b1)


def _fin_body(g0_ref, g1_ref, hp_ref, d0_ref, d1_ref, b2_ref, w2_ref, o_ref):
    dinv = lax.rsqrt(d0_ref[...] + d1_ref[...] + 1.0)
    t = (g0_ref[...] + g1_ref[...] + hp_ref[...]) * dinv
    o_ref[...] = jnp.dot(t, w2_ref[...], precision=lax.Precision.HIGHEST,
                         preferred_element_type=jnp.float32) + b2_ref[...]


def _fin(g0, g1, hp, deg0, deg1, b2, W2):
    return pl.pallas_call(
        _fin_body,
        grid=(_GRID,),
        in_specs=[
            pl.BlockSpec((_GB, _HID), lambda i: (i, 0)),
            pl.BlockSpec((_GB, _HID), lambda i: (i, 0)),
            pl.BlockSpec((_GB, _HID), lambda i: (i, 0)),
            pl.BlockSpec((_GB, 1), lambda i: (i, 0)),
            pl.BlockSpec((_GB, 1), lambda i: (i, 0)),
            pl.BlockSpec((1, _OUT), lambda i: (0, 0)),
            pl.BlockSpec((_HID, _OUT), lambda i: (0, 0)),
        ],
        out_specs=pl.BlockSpec((_GB, _OUT), lambda i: (i, 0)),
        out_shape=jax.ShapeDtypeStruct((_N, _OUT), jnp.float32),
    )(g0, g1, hp, deg0, deg1, b2, W2)


def kernel(x, edge_index, W1, b1, W2, b2):
    src = edge_index[0]
    dst = edge_index[1]
    dstd = edge_index[1].reshape(_NW, _DNC, _DCH)

    degp = _deg_kernel(dstd).reshape(_NC, _ND)
    deg0 = degp[0, :_N, None]
    deg1 = degp[1, :_N, None]

    h1p = _mm1(x, W1, deg0, deg1)
    agg1 = _agg64(h1p, src, dst)
    h2p = _mid(agg1[0], agg1[1], h1p, deg0, deg1,
               b1.reshape(1, _HID))
    agg2 = _agg64(h2p, src, dst)
    return _fin(agg2[0], agg2[1], h2p, deg0, deg1,
                b2.reshape(1, _OUT), W2)


# speedup vs baseline: 1.0052x; 1.0002x over previous
import functools
import jax
import jax.numpy as jnp
from jax import lax
from jax.experimental import pallas as pl
from jax.experimental.pallas import tpu as pltpu
from jax.experimental.pallas import tpu_sc as plsc

_N = 10000
_E = 320000
_IN = 128
_HID = 64
_OUT = 128

_NC = 2
_NS = 16
_NW = _NC * _NS
_EPW = _E // _NW
_CH = 128
_NF = _EPW // _CH
_REM = _EPW - _NF * _CH
_RBASE = _NF * _CH
_DCH = 80
_DNC = _EPW // _DCH

_ND = 10240
_DPT = _ND // _NS

_CT = 10
_RPT = _N // _CT

_mesh = plsc.VectorSubcoreMesh(core_axis_name="c", subcore_axis_name="s")


@functools.partial(
    pl.kernel,
    out_type=jax.ShapeDtypeStruct((_NC * _ND,), jnp.float32),
    mesh=_mesh,
    compiler_params=pltpu.CompilerParams(skip_device_barrier=True),
    scratch_types=[
        pltpu.VMEM_SHARED((_ND,), jnp.float32),
        pltpu.VMEM((_DNC, _DCH), jnp.int32),
        pltpu.VMEM((_DCH,), jnp.float32),
        pltpu.VMEM((_DPT,), jnp.float32),
        pltpu.SemaphoreType.DMA,
        pltpu.SemaphoreType.DMA,
    ],
)
def _deg_kernel(dst_hbm, out_hbm, acc, didx, ones_v, zb, isem, dsem):
    c = lax.axis_index("c")
    s = lax.axis_index("s")
    wid = c * _NS + s

    icopy = pltpu.async_copy(dst_hbm.at[wid], didx, isem)

    def fill(i, _):
        zb[pl.ds(i * 16, 16)] = jnp.zeros((16,), jnp.float32)
        return 0
    lax.fori_loop(0, _DPT // 16, fill, 0)
    for k in range(_DCH // 16):
        ones_v[pl.ds(k * 16, 16)] = jnp.ones((16,), jnp.float32)

    pltpu.sync_copy(zb, acc.at[pl.ds(s * _DPT, _DPT)])
    plsc.subcore_barrier()
    icopy.wait()

    def dscat(j):
        pltpu.async_copy(ones_v, acc.at[didx.at[j]], dsem, add=True)

    def dwait(j):
        pltpu.make_async_copy(ones_v, acc.at[didx.at[j]], dsem).wait()

    for b in range(5):
        dscat(b)

    def dgroup(g, _):
        for b in range(5):
            dscat((g + 1) * 5 + b)
        for b in range(5):
            dwait(g * 5 + b)
        return 0
    lax.fori_loop(0, _DNC // 5 - 1, dgroup, 0)
    for b in range(5):
        dwait(_DNC - 5 + b)

    plsc.subcore_barrier()
    pltpu.sync_copy(acc.at[pl.ds(s * _DPT, _DPT)],
                    out_hbm.at[pl.ds(c * _ND + s * _DPT, _DPT)])


def _make_agg(D):
    @functools.partial(
        pl.kernel,
        out_type=jax.ShapeDtypeStruct((_NC, _N, D), jnp.float32),
        mesh=_mesh,
        compiler_params=pltpu.CompilerParams(use_tc_tiling_on_sc=False, skip_device_barrier=True),
        scratch_types=[
            pltpu.VMEM_SHARED((_N, D), jnp.float32),
            pltpu.VMEM((_EPW,), jnp.int32),
            pltpu.VMEM((_EPW,), jnp.int32),
            pltpu.VMEM((_CH, D), jnp.float32),
            pltpu.VMEM((_CH, D), jnp.float32),
            pltpu.VMEM((_CH, D), jnp.float32),
            pltpu.VMEM((_CH, D), jnp.float32),
            pltpu.SemaphoreType.DMA,
            pltpu.SemaphoreType.DMA,
            pltpu.SemaphoreType.DMA,
            pltpu.SemaphoreType.DMA,
            pltpu.SemaphoreType.DMA,
            pltpu.SemaphoreType.DMA,
            pltpu.SemaphoreType.DMA,
            pltpu.SemaphoreType.DMA,
            pltpu.SemaphoreType.DMA,
        ],
    )
    def _agg(hp_hbm, src_hbm, dst_hbm, out_hbm, acc, sidx, dall,
             r0, r1, r2, r3,
             isem, g0, g1, g2, g3, s0, s1, s2, s3):
        rows = [r0, r1, r2, r3]
        gs = [g0, g1, g2, g3]
        ss = [s0, s1, s2, s3]
        c = lax.axis_index("c")
        s = lax.axis_index("s")
        wid = c * _NS + s
        ebase = wid * _EPW

        ic1 = pltpu.async_copy(src_hbm.at[pl.ds(ebase, _EPW)], sidx, isem)
        ic2 = pltpu.async_copy(dst_hbm.at[pl.ds(ebase, _EPW)], dall, isem)

        def zrow(r, _):
            for j in range(D // 16):
                r0[r, pl.ds(j * 16, 16)] = jnp.zeros((16,), jnp.float32)
            return 0
        lax.fori_loop(0, 40, zrow, 0)

        @pl.when(s < _CT)
        def _zero():
            zc = 40
            nz = _RPT // zc
            for t0 in range(0, nz, 5):
                for t in range(t0, min(t0 + 5, nz)):
                    pltpu.async_copy(
                        r0.at[pl.ds(0, zc)],
                        acc.at[pl.ds(s * _RPT + t * zc, zc)], g0)
                for t in range(t0, min(t0 + 5, nz)):
                    pltpu.make_async_copy(
                        r0.at[pl.ds(0, zc)],
                        acc.at[pl.ds(s * _RPT + t * zc, zc)], g0).wait()

        plsc.subcore_barrier()
        ic1.wait()
        ic2.wait()

        def gather(j, b):
            pltpu.async_copy(hp_hbm.at[sidx.at[pl.ds(j * _CH, _CH)]],
                             rows[b], gs[b])

        def wait_gather(j, b):
            pltpu.make_async_copy(hp_hbm.at[sidx.at[pl.ds(j * _CH, _CH)]],
                                  rows[b], gs[b]).wait()

        def scatter(j, b):
            pltpu.async_copy(rows[b], acc.at[dall.at[pl.ds(j * _CH, _CH)]],
                             ss[b], add=True)

        def wait_scatter(j, b):
            pltpu.make_async_copy(rows[b],
                                  acc.at[dall.at[pl.ds(j * _CH, _CH)]],
                                  ss[b]).wait()

        gather(0, 0)
        gather(1, 1)
        for j in (0, 1):
            gather(j + 2, j + 2)
            wait_gather(j, j)
            scatter(j, j)

        def group(g, _):
            for b in range(4):
                j = 2 + g * 4 + b
                jb = (2 + b) % 4
                wait_scatter(j - 2, b)
                gather(j + 2, b)
                wait_gather(j, jb)
                scatter(j, jb)
            return 0
        lax.fori_loop(0, (_NF - 6) // 4, group, 0)

        jt = _NF - 4
        wait_scatter(jt - 2, 0)
        gather(jt + 2, 0)
        wait_gather(jt, 2)
        scatter(jt, 2)
        wait_scatter(jt - 1, 1)
        gather(jt + 3, 1)
        wait_gather(jt + 1, 3)
        scatter(jt + 1, 3)
        wait_gather(jt + 2, 0)
        scatter(jt + 2, 0)
        wait_gather(jt + 3, 1)
        scatter(jt + 3, 1)
        wait_scatter(jt, 2)
        wait_scatter(jt + 1, 3)
        wait_scatter(jt + 2, 0)
        wait_scatter(jt + 3, 1)

        rsl = r2.at[pl.ds(0, _REM)]
        pltpu.async_copy(hp_hbm.at[sidx.at[pl.ds(_RBASE, _REM)]], rsl, g2)
        pltpu.make_async_copy(
            hp_hbm.at[sidx.at[pl.ds(_RBASE, _REM)]], rsl, g2).wait()
        pltpu.async_copy(rsl, acc.at[dall.at[pl.ds(_RBASE, _REM)]],
                         s2, add=True)
        pltpu.make_async_copy(rsl, acc.at[dall.at[pl.ds(_RBASE, _REM)]],
                              s2).wait()

        plsc.subcore_barrier()

        @pl.when(s < _CT)
        def _copy_out():
            pltpu.sync_copy(acc.at[pl.ds(s * _RPT, _RPT)],
                            out_hbm.at[c, pl.ds(s * _RPT, _RPT)])
    return _agg


_agg64 = _make_agg(_HID)


_GB = 1000
_GRID = _N // _GB


def _mm1_body(x_ref, w_ref, d0_ref, d1_ref, o_ref):
    dinv = lax.rsqrt(d0_ref[...] + d1_ref[...] + 1.0)
    o_ref[...] = jnp.dot(x_ref[...], w_ref[...],
                         preferred_element_type=jnp.float32) * dinv


def _mm1(x, W1, deg0, deg1):
    return pl.pallas_call(
        _mm1_body,
        grid=(_GRID,),
        in_specs=[
            pl.BlockSpec((_GB, _IN), lambda i: (i, 0)),
            pl.BlockSpec((_IN, _HID), lambda i: (0, 0)),
            pl.BlockSpec((_GB, 1), lambda i: (i, 0)),
            pl.BlockSpec((_GB, 1), lambda i: (i, 0)),
        ],
        out_specs=pl.BlockSpec((_GB, _HID), lambda i: (i, 0)),
        out_shape=jax.ShapeDtypeStruct((_N, _HID), jnp.float32),
    )(x, W1, deg0, deg1)


def _mid_body(a0_ref, a1_ref, hp_ref, d0_ref, d1_ref, b1_ref, o_ref):
    dinv = lax.rsqrt(d0_ref[...] + d1_ref[...] + 1.0)
    acc = a0_ref[...] + a1_ref[...] + hp_ref[...]
    o_ref[...] = jnp.maximum(acc * dinv + b1_ref[...], 0.0) * dinv


def _mid(a0, a1, hp, deg0, deg1, b1):
    return pl.pallas_call(
        _mid_body,
        grid=(_GRID,),
        in_specs=[
            pl.BlockSpec((_GB, _HID), lambda i: (i, 0)),
            pl.BlockSpec((_GB, _HID), lambda i: (i, 0)),
            pl.BlockSpec((_GB, _HID), lambda i: (i, 0)),
            pl.BlockSpec((_GB, 1), lambda i: (i, 0)),
            pl.BlockSpec((_GB, 1), lambda i: (i, 0)),
            pl.BlockSpec((1, _HID), lambda i: (0, 0)),
        ],
        out_specs=pl.BlockSpec((_GB, _HID), lambda i: (i, 0)),
        out_shape=jax.ShapeDtypeStruct((_N, _HID), jnp.float32),
    )(a0, a1, hp, deg0, deg1, b1)


def _fin_body(g0_ref, g1_ref, hp_ref, d0_ref, d1_ref, b2_ref, w2_ref, o_ref):
    dinv = lax.rsqrt(d0_ref[...] + d1_ref[...] + 1.0)
    t = (g0_ref[...] + g1_ref[...] + hp_ref[...]) * dinv
    o_ref[...] = jnp.dot(t, w2_ref[...], precision=lax.Precision.HIGHEST,
                         preferred_element_type=jnp.float32) + b2_ref[...]


def _fin(g0, g1, hp, deg0, deg1, b2, W2):
    return pl.pallas_call(
        _fin_body,
        grid=(_GRID,),
        in_specs=[
            pl.BlockSpec((_GB, _HID), lambda i: (i, 0)),
            pl.BlockSpec((_GB, _HID), lambda i: (i, 0)),
            pl.BlockSpec((_GB, _HID), lambda i: (i, 0)),
            pl.BlockSpec((_GB, 1), lambda i: (i, 0)),
            pl.BlockSpec((_GB, 1), lambda i: (i, 0)),
            pl.BlockSpec((1, _OUT), lambda i: (0, 0)),
            pl.BlockSpec((_HID, _OUT), lambda i: (0, 0)),
        ],
        out_specs=pl.BlockSpec((_GB, _OUT), lambda i: (i, 0)),
        out_shape=jax.ShapeDtypeStruct((_N, _OUT), jnp.float32),
    )(g0, g1, hp, deg0, deg1, b2, W2)


def kernel(x, edge_index, W1, b1, W2, b2):
    src = edge_index[0]
    dst = edge_index[1]
    dstd = edge_index[1].reshape(_NW, _DNC, _DCH)

    degp = _deg_kernel(dstd).reshape(_NC, _ND)
    deg0 = degp[0, :_N, None]
    deg1 = degp[1, :_N, None]

    h1p = _mm1(x, W1, deg0, deg1)
    agg1 = _agg64(h1p, src, dst)
    h2p = _mid(agg1[0], agg1[1], h1p, deg0, deg1,
               b1.reshape(1, _HID))
    agg2 = _agg64(h2p, src, dst)
    return _fin(agg2[0], agg2[1], h2p, deg0, deg1,
                b2.reshape(1, _OUT), W2)


# speedup vs baseline: 1.1061x; 1.1003x over previous
import functools
import jax
import jax.numpy as jnp
from jax import lax
from jax.experimental import pallas as pl
from jax.experimental.pallas import tpu as pltpu
from jax.experimental.pallas import tpu_sc as plsc

_N = 10000
_E = 320000
_IN = 128
_HID = 64
_OUT = 128

_NC = 2
_NS = 16
_NW = _NC * _NS
_EPW = _E // _NW
_CH = 128
_NF = _EPW // _CH
_REM = _EPW - _NF * _CH
_RBASE = _NF * _CH
_DCH = 80
_DNC = _EPW // _DCH

_ND = 10240
_DPT = _ND // _NS

_CT = 10
_RPT = _N // _CT

_mesh = plsc.VectorSubcoreMesh(core_axis_name="c", subcore_axis_name="s")


@functools.partial(
    pl.kernel,
    out_type=jax.ShapeDtypeStruct((_NC * _ND,), jnp.float32),
    mesh=_mesh,
    compiler_params=pltpu.CompilerParams(use_tc_tiling_on_sc=False),
    scratch_types=[
        pltpu.VMEM_SHARED((_ND,), jnp.float32),
        pltpu.VMEM((_EPW,), jnp.int32),
        pltpu.VMEM((_DCH,), jnp.float32),
        pltpu.VMEM((_DPT,), jnp.float32),
        pltpu.SemaphoreType.DMA,
        pltpu.SemaphoreType.DMA,
    ],
)
def _deg_kernel(dst_hbm, out_hbm, acc, dall, ones_v, zb, isem, dsem):
    c = lax.axis_index("c")
    s = lax.axis_index("s")
    wid = c * _NS + s
    ebase = wid * _EPW

    icopy = pltpu.async_copy(dst_hbm.at[pl.ds(ebase, _EPW)], dall, isem)

    def fill(i, _):
        zb[pl.ds(i * 16, 16)] = jnp.zeros((16,), jnp.float32)
        return 0
    lax.fori_loop(0, _DPT // 16, fill, 0)
    for k in range(_DCH // 16):
        ones_v[pl.ds(k * 16, 16)] = jnp.ones((16,), jnp.float32)

    pltpu.sync_copy(zb, acc.at[pl.ds(s * _DPT, _DPT)])
    plsc.subcore_barrier()
    icopy.wait()

    def dscat(j):
        pltpu.async_copy(ones_v, acc.at[dall.at[pl.ds(j * _DCH, _DCH)]],
                         dsem, add=True)

    def dwait(j):
        pltpu.make_async_copy(ones_v, acc.at[dall.at[pl.ds(j * _DCH, _DCH)]],
                              dsem).wait()

    for b in range(5):
        dscat(b)

    def dgroup(g, _):
        for b in range(5):
            dscat((g + 1) * 5 + b)
        for b in range(5):
            dwait(g * 5 + b)
        return 0
    lax.fori_loop(0, _DNC // 5 - 1, dgroup, 0)
    for b in range(5):
        dwait(_DNC - 5 + b)

    plsc.subcore_barrier()
    pltpu.sync_copy(acc.at[pl.ds(s * _DPT, _DPT)],
                    out_hbm.at[pl.ds(c * _ND + s * _DPT, _DPT)])


def _make_agg(D):
    @functools.partial(
        pl.kernel,
        out_type=jax.ShapeDtypeStruct((_NC, _N, D), jnp.float32),
        mesh=_mesh,
        compiler_params=pltpu.CompilerParams(use_tc_tiling_on_sc=False),
        scratch_types=[
            pltpu.VMEM_SHARED((_N, D), jnp.float32),
            pltpu.VMEM((_EPW,), jnp.int32),
            pltpu.VMEM((_EPW,), jnp.int32),
            pltpu.VMEM((_CH, D), jnp.float32),
            pltpu.VMEM((_CH, D), jnp.float32),
            pltpu.VMEM((_CH, D), jnp.float32),
            pltpu.VMEM((_CH, D), jnp.float32),
            pltpu.SemaphoreType.DMA,
            pltpu.SemaphoreType.DMA,
            pltpu.SemaphoreType.DMA,
            pltpu.SemaphoreType.DMA,
            pltpu.SemaphoreType.DMA,
            pltpu.SemaphoreType.DMA,
            pltpu.SemaphoreType.DMA,
            pltpu.SemaphoreType.DMA,
            pltpu.SemaphoreType.DMA,
        ],
    )
    def _agg(hp_hbm, src_hbm, dst_hbm, out_hbm, acc, sidx, dall,
             r0, r1, r2, r3,
             isem, g0, g1, g2, g3, s0, s1, s2, s3):
        rows = [r0, r1, r2, r3]
        gs = [g0, g1, g2, g3]
        ss = [s0, s1, s2, s3]
        c = lax.axis_index("c")
        s = lax.axis_index("s")
        wid = c * _NS + s
        ebase = wid * _EPW

        ic1 = pltpu.async_copy(src_hbm.at[pl.ds(ebase, _EPW)], sidx, isem)
        ic2 = pltpu.async_copy(dst_hbm.at[pl.ds(ebase, _EPW)], dall, isem)

        def zrow(r, _):
            for j in range(D // 16):
                r0[r, pl.ds(j * 16, 16)] = jnp.zeros((16,), jnp.float32)
            return 0
        lax.fori_loop(0, 40, zrow, 0)

        @pl.when(s < _CT)
        def _zero():
            zc = 40
            nz = _RPT // zc
            for t0 in range(0, nz, 5):
                for t in range(t0, min(t0 + 5, nz)):
                    pltpu.async_copy(
                        r0.at[pl.ds(0, zc)],
                        acc.at[pl.ds(s * _RPT + t * zc, zc)], g0)
                for t in range(t0, min(t0 + 5, nz)):
                    pltpu.make_async_copy(
                        r0.at[pl.ds(0, zc)],
                        acc.at[pl.ds(s * _RPT + t * zc, zc)], g0).wait()

        plsc.subcore_barrier()
        ic1.wait()
        ic2.wait()

        def gather(j, b):
            pltpu.async_copy(hp_hbm.at[sidx.at[pl.ds(j * _CH, _CH)]],
                             rows[b], gs[b])

        def wait_gather(j, b):
            pltpu.make_async_copy(hp_hbm.at[sidx.at[pl.ds(j * _CH, _CH)]],
                                  rows[b], gs[b]).wait()

        def scatter(j, b):
            pltpu.async_copy(rows[b], acc.at[dall.at[pl.ds(j * _CH, _CH)]],
                             ss[b], add=True)

        def wait_scatter(j, b):
            pltpu.make_async_copy(rows[b],
                                  acc.at[dall.at[pl.ds(j * _CH, _CH)]],
                                  ss[b]).wait()

        gather(0, 0)
        gather(1, 1)
        for j in (0, 1):
            gather(j + 2, j + 2)
            wait_gather(j, j)
            scatter(j, j)

        def group(g, _):
            for b in range(4):
                j = 2 + g * 4 + b
                jb = (2 + b) % 4
                wait_scatter(j - 2, b)
                gather(j + 2, b)
                wait_gather(j, jb)
                scatter(j, jb)
            return 0
        lax.fori_loop(0, (_NF - 6) // 4, group, 0)

        jt = _NF - 4
        wait_scatter(jt - 2, 0)
        gather(jt + 2, 0)
        wait_gather(jt, 2)
        scatter(jt, 2)
        wait_scatter(jt - 1, 1)
        gather(jt + 3, 1)
        wait_gather(jt + 1, 3)
        scatter(jt + 1, 3)
        wait_gather(jt + 2, 0)
        scatter(jt + 2, 0)
        wait_gather(jt + 3, 1)
        scatter(jt + 3, 1)
        wait_scatter(jt, 2)
        wait_scatter(jt + 1, 3)
        wait_scatter(jt + 2, 0)
        wait_scatter(jt + 3, 1)

        rsl = r2.at[pl.ds(0, _REM)]
        pltpu.async_copy(hp_hbm.at[sidx.at[pl.ds(_RBASE, _REM)]], rsl, g2)
        pltpu.make_async_copy(
            hp_hbm.at[sidx.at[pl.ds(_RBASE, _REM)]], rsl, g2).wait()
        pltpu.async_copy(rsl, acc.at[dall.at[pl.ds(_RBASE, _REM)]],
                         s2, add=True)
        pltpu.make_async_copy(rsl, acc.at[dall.at[pl.ds(_RBASE, _REM)]],
                              s2).wait()

        plsc.subcore_barrier()

        @pl.when(s < _CT)
        def _copy_out():
            pltpu.sync_copy(acc.at[pl.ds(s * _RPT, _RPT)],
                            out_hbm.at[c, pl.ds(s * _RPT, _RPT)])
    return _agg


_agg64 = _make_agg(_HID)


_GB = 5000
_GRID = _N // _GB


def _mm1_body(x_ref, w_ref, d0_ref, d1_ref, o_ref):
    dinv = lax.rsqrt(d0_ref[...] + d1_ref[...] + 1.0)
    o_ref[...] = jnp.dot(x_ref[...], w_ref[...],
                         preferred_element_type=jnp.float32) * dinv


def _mm1(x, W1, deg0, deg1):
    return pl.pallas_call(
        _mm1_body,
        grid=(_GRID,),
        in_specs=[
            pl.BlockSpec((_GB, _IN), lambda i: (i, 0)),
            pl.BlockSpec((_IN, _HID), lambda i: (0, 0)),
            pl.BlockSpec((_GB, 1), lambda i: (i, 0)),
            pl.BlockSpec((_GB, 1), lambda i: (i, 0)),
        ],
        out_specs=pl.BlockSpec((_GB, _HID), lambda i: (i, 0)),
        out_shape=jax.ShapeDtypeStruct((_N, _HID), jnp.float32),
    )(x, W1, deg0, deg1)


def _mid_body(a_ref, hp_ref, d0_ref, d1_ref, b1_ref, o_ref):
    dinv = lax.rsqrt(d0_ref[...] + d1_ref[...] + 1.0)
    acc = a_ref[0] + a_ref[1] + hp_ref[...]
    o_ref[...] = jnp.maximum(acc * dinv + b1_ref[...], 0.0) * dinv


def _mid(a, hp, deg0, deg1, b1):
    return pl.pallas_call(
        _mid_body,
        grid=(_GRID,),
        in_specs=[
            pl.BlockSpec((_NC, _GB, _HID), lambda i: (0, i, 0)),
            pl.BlockSpec((_GB, _HID), lambda i: (i, 0)),
            pl.BlockSpec((_GB, 1), lambda i: (i, 0)),
            pl.BlockSpec((_GB, 1), lambda i: (i, 0)),
            pl.BlockSpec((1, _HID), lambda i: (0, 0)),
        ],
        out_specs=pl.BlockSpec((_GB, _HID), lambda i: (i, 0)),
        out_shape=jax.ShapeDtypeStruct((_N, _HID), jnp.float32),
    )(a, hp, deg0, deg1, b1)


def _fin_body(g_ref, hp_ref, d0_ref, d1_ref, b2_ref, w2_ref, o_ref):
    dinv = lax.rsqrt(d0_ref[...] + d1_ref[...] + 1.0)
    t = (g_ref[0] + g_ref[1] + hp_ref[...]) * dinv
    o_ref[...] = jnp.dot(t, w2_ref[...], precision=lax.Precision.HIGHEST,
                         preferred_element_type=jnp.float32) + b2_ref[...]


def _fin(g, hp, deg0, deg1, b2, W2):
    return pl.pallas_call(
        _fin_body,
        grid=(_GRID,),
        in_specs=[
            pl.BlockSpec((_NC, _GB, _HID), lambda i: (0, i, 0)),
            pl.BlockSpec((_GB, _HID), lambda i: (i, 0)),
            pl.BlockSpec((_GB, 1), lambda i: (i, 0)),
            pl.BlockSpec((_GB, 1), lambda i: (i, 0)),
            pl.BlockSpec((1, _OUT), lambda i: (0, 0)),
            pl.BlockSpec((_HID, _OUT), lambda i: (0, 0)),
        ],
        out_specs=pl.BlockSpec((_GB, _OUT), lambda i: (i, 0)),
        out_shape=jax.ShapeDtypeStruct((_N, _OUT), jnp.float32),
    )(g, hp, deg0, deg1, b2, W2)


def kernel(x, edge_index, W1, b1, W2, b2):
    src = edge_index[0]
    dst = edge_index[1]

    degp = _deg_kernel(dst).reshape(_NC, _ND)
    deg0 = degp[0, :_N, None]
    deg1 = degp[1, :_N, None]

    h1p = _mm1(x, W1, deg0, deg1)
    agg1 = _agg64(h1p, src, dst)
    h2p = _mid(agg1, h1p, deg0, deg1,
               b1.reshape(1, _HID))
    agg2 = _agg64(h2p, src, dst)
    return _fin(agg2, h2p, deg0, deg1,
                b2.reshape(1, _OUT), W2)


# speedup vs baseline: 1.1896x; 1.0755x over previous
import functools
import jax
import jax.numpy as jnp
from jax import lax
from jax.experimental import pallas as pl
from jax.experimental.pallas import tpu as pltpu
from jax.experimental.pallas import tpu_sc as plsc

_N = 10000
_E = 320000
_IN = 128
_HID = 64
_OUT = 128

_NC = 2
_NS = 16
_NW = _NC * _NS
_EPW = _E // _NW
_CH = 128
_NF = _EPW // _CH
_REM = _EPW - _NF * _CH
_RBASE = _NF * _CH
_DCH = 80
_DNC = _EPW // _DCH

_ND = 10240
_DPT = _ND // _NS

_CT = 10
_RPT = _N // _CT

_mesh = plsc.VectorSubcoreMesh(core_axis_name="c", subcore_axis_name="s")


@functools.partial(
    pl.kernel,
    out_type=jax.ShapeDtypeStruct((_NC * _ND,), jnp.float32),
    mesh=_mesh,
    compiler_params=pltpu.CompilerParams(use_tc_tiling_on_sc=False),
    scratch_types=[
        pltpu.VMEM_SHARED((_ND,), jnp.float32),
        pltpu.VMEM((_EPW,), jnp.int32),
        pltpu.VMEM((_DCH,), jnp.float32),
        pltpu.VMEM((_DPT,), jnp.float32),
        pltpu.SemaphoreType.DMA,
        pltpu.SemaphoreType.DMA,
    ],
)
def _deg_kernel(ei_hbm, out_hbm, acc, dall, ones_v, zb, isem, dsem):
    c = lax.axis_index("c")
    s = lax.axis_index("s")
    wid = c * _NS + s
    ebase = wid * _EPW

    icopy = pltpu.async_copy(ei_hbm.at[1, pl.ds(ebase, _EPW)], dall, isem)

    def fill(i, _):
        zb[pl.ds(i * 16, 16)] = jnp.zeros((16,), jnp.float32)
        return 0
    lax.fori_loop(0, _DPT // 16, fill, 0)
    for k in range(_DCH // 16):
        ones_v[pl.ds(k * 16, 16)] = jnp.ones((16,), jnp.float32)

    pltpu.sync_copy(zb, acc.at[pl.ds(s * _DPT, _DPT)])
    plsc.subcore_barrier()
    icopy.wait()

    def dscat(j):
        pltpu.async_copy(ones_v, acc.at[dall.at[pl.ds(j * _DCH, _DCH)]],
                         dsem, add=True)

    def dwait(j):
        pltpu.make_async_copy(ones_v, acc.at[dall.at[pl.ds(j * _DCH, _DCH)]],
                              dsem).wait()

    for b in range(5):
        dscat(b)

    def dgroup(g, _):
        for b in range(5):
            dscat((g + 1) * 5 + b)
        for b in range(5):
            dwait(g * 5 + b)
        return 0
    lax.fori_loop(0, _DNC // 5 - 1, dgroup, 0)
    for b in range(5):
        dwait(_DNC - 5 + b)

    plsc.subcore_barrier()
    pltpu.sync_copy(acc.at[pl.ds(s * _DPT, _DPT)],
                    out_hbm.at[pl.ds(c * _ND + s * _DPT, _DPT)])


def _make_agg(D):
    @functools.partial(
        pl.kernel,
        out_type=jax.ShapeDtypeStruct((_NC, _N, D), jnp.float32),
        mesh=_mesh,
        compiler_params=pltpu.CompilerParams(use_tc_tiling_on_sc=False),
        scratch_types=[
            pltpu.VMEM_SHARED((_N, D), jnp.float32),
            pltpu.VMEM((_EPW,), jnp.int32),
            pltpu.VMEM((_EPW,), jnp.int32),
            pltpu.VMEM((_CH, D), jnp.float32),
            pltpu.VMEM((_CH, D), jnp.float32),
            pltpu.VMEM((_CH, D), jnp.float32),
            pltpu.VMEM((_CH, D), jnp.float32),
            pltpu.SemaphoreType.DMA,
            pltpu.SemaphoreType.DMA,
            pltpu.SemaphoreType.DMA,
            pltpu.SemaphoreType.DMA,
            pltpu.SemaphoreType.DMA,
            pltpu.SemaphoreType.DMA,
            pltpu.SemaphoreType.DMA,
            pltpu.SemaphoreType.DMA,
            pltpu.SemaphoreType.DMA,
        ],
    )
    def _agg(hp_hbm, ei_hbm, out_hbm, acc, sidx, dall,
             r0, r1, r2, r3,
             isem, g0, g1, g2, g3, s0, s1, s2, s3):
        rows = [r0, r1, r2, r3]
        gs = [g0, g1, g2, g3]
        ss = [s0, s1, s2, s3]
        c = lax.axis_index("c")
        s = lax.axis_index("s")
        wid = c * _NS + s
        ebase = wid * _EPW

        ic1 = pltpu.async_copy(ei_hbm.at[0, pl.ds(ebase, _EPW)], sidx, isem)
        ic2 = pltpu.async_copy(ei_hbm.at[1, pl.ds(ebase, _EPW)], dall, isem)

        def zrow(r, _):
            for j in range(D // 16):
                r0[r, pl.ds(j * 16, 16)] = jnp.zeros((16,), jnp.float32)
            return 0
        lax.fori_loop(0, 40, zrow, 0)

        @pl.when(s < _CT)
        def _zero():
            zc = 40
            nz = _RPT // zc
            for t0 in range(0, nz, 5):
                for t in range(t0, min(t0 + 5, nz)):
                    pltpu.async_copy(
                        r0.at[pl.ds(0, zc)],
                        acc.at[pl.ds(s * _RPT + t * zc, zc)], g0)
                for t in range(t0, min(t0 + 5, nz)):
                    pltpu.make_async_copy(
                        r0.at[pl.ds(0, zc)],
                        acc.at[pl.ds(s * _RPT + t * zc, zc)], g0).wait()

        plsc.subcore_barrier()
        ic1.wait()
        ic2.wait()

        def gather(j, b):
            pltpu.async_copy(hp_hbm.at[sidx.at[pl.ds(j * _CH, _CH)]],
                             rows[b], gs[b])

        def wait_gather(j, b):
            pltpu.make_async_copy(hp_hbm.at[sidx.at[pl.ds(j * _CH, _CH)]],
                                  rows[b], gs[b]).wait()

        def scatter(j, b):
            pltpu.async_copy(rows[b], acc.at[dall.at[pl.ds(j * _CH, _CH)]],
                             ss[b], add=True)

        def wait_scatter(j, b):
            pltpu.make_async_copy(rows[b],
                                  acc.at[dall.at[pl.ds(j * _CH, _CH)]],
                                  ss[b]).wait()

        gather(0, 0)
        gather(1, 1)
        for j in (0, 1):
            gather(j + 2, j + 2)
            wait_gather(j, j)
            scatter(j, j)

        def group(g, _):
            for b in range(4):
                j = 2 + g * 4 + b
                jb = (2 + b) % 4
                wait_scatter(j - 2, b)
                gather(j + 2, b)
                wait_gather(j, jb)
                scatter(j, jb)
            return 0
        lax.fori_loop(0, (_NF - 6) // 4, group, 0)

        jt = _NF - 4
        wait_scatter(jt - 2, 0)
        gather(jt + 2, 0)
        wait_gather(jt, 2)
        scatter(jt, 2)
        wait_scatter(jt - 1, 1)
        gather(jt + 3, 1)
        wait_gather(jt + 1, 3)
        scatter(jt + 1, 3)
        wait_gather(jt + 2, 0)
        scatter(jt + 2, 0)
        wait_gather(jt + 3, 1)
        scatter(jt + 3, 1)
        wait_scatter(jt, 2)
        wait_scatter(jt + 1, 3)
        wait_scatter(jt + 2, 0)
        wait_scatter(jt + 3, 1)

        rsl = r2.at[pl.ds(0, _REM)]
        pltpu.async_copy(hp_hbm.at[sidx.at[pl.ds(_RBASE, _REM)]], rsl, g2)
        pltpu.make_async_copy(
            hp_hbm.at[sidx.at[pl.ds(_RBASE, _REM)]], rsl, g2).wait()
        pltpu.async_copy(rsl, acc.at[dall.at[pl.ds(_RBASE, _REM)]],
                         s2, add=True)
        pltpu.make_async_copy(rsl, acc.at[dall.at[pl.ds(_RBASE, _REM)]],
                              s2).wait()

        plsc.subcore_barrier()

        @pl.when(s < _CT)
        def _copy_out():
            pltpu.sync_copy(acc.at[pl.ds(s * _RPT, _RPT)],
                            out_hbm.at[c, pl.ds(s * _RPT, _RPT)])
    return _agg


_agg64 = _make_agg(_HID)


_GB = 5000
_GRID = _N // _GB


def _mm1_body(x_ref, w_ref, d0_ref, d1_ref, o_ref):
    dinv = lax.rsqrt(d0_ref[...] + d1_ref[...] + 1.0)
    o_ref[...] = jnp.dot(x_ref[...], w_ref[...],
                         preferred_element_type=jnp.float32) * dinv


def _mm1(x, W1, deg0, deg1):
    return pl.pallas_call(
        _mm1_body,
        grid=(_GRID,),
        in_specs=[
            pl.BlockSpec((_GB, _IN), lambda i: (i, 0)),
            pl.BlockSpec((_IN, _HID), lambda i: (0, 0)),
            pl.BlockSpec((_GB, 1), lambda i: (i, 0)),
            pl.BlockSpec((_GB, 1), lambda i: (i, 0)),
        ],
        out_specs=pl.BlockSpec((_GB, _HID), lambda i: (i, 0)),
        out_shape=jax.ShapeDtypeStruct((_N, _HID), jnp.float32),
    )(x, W1, deg0, deg1)


def _mid_body(a_ref, hp_ref, d0_ref, d1_ref, b1_ref, o_ref):
    dinv = lax.rsqrt(d0_ref[...] + d1_ref[...] + 1.0)
    acc = a_ref[0] + a_ref[1] + hp_ref[...]
    o_ref[...] = jnp.maximum(acc * dinv + b1_ref[...], 0.0) * dinv


def _mid(a, hp, deg0, deg1, b1):
    return pl.pallas_call(
        _mid_body,
        grid=(_GRID,),
        in_specs=[
            pl.BlockSpec((_NC, _GB, _HID), lambda i: (0, i, 0)),
            pl.BlockSpec((_GB, _HID), lambda i: (i, 0)),
            pl.BlockSpec((_GB, 1), lambda i: (i, 0)),
            pl.BlockSpec((_GB, 1), lambda i: (i, 0)),
            pl.BlockSpec((1, _HID), lambda i: (0, 0)),
        ],
        out_specs=pl.BlockSpec((_GB, _HID), lambda i: (i, 0)),
        out_shape=jax.ShapeDtypeStruct((_N, _HID), jnp.float32),
    )(a, hp, deg0, deg1, b1)


def _fin_body(g_ref, hp_ref, d0_ref, d1_ref, b2_ref, w2_ref, o_ref):
    dinv = lax.rsqrt(d0_ref[...] + d1_ref[...] + 1.0)
    t = (g_ref[0] + g_ref[1] + hp_ref[...]) * dinv
    o_ref[...] = jnp.dot(t, w2_ref[...], preferred_element_type=jnp.float32) + b2_ref[...]


def _fin(g, hp, deg0, deg1, b2, W2):
    return pl.pallas_call(
        _fin_body,
        grid=(_GRID,),
        in_specs=[
            pl.BlockSpec((_NC, _GB, _HID), lambda i: (0, i, 0)),
            pl.BlockSpec((_GB, _HID), lambda i: (i, 0)),
            pl.BlockSpec((_GB, 1), lambda i: (i, 0)),
            pl.BlockSpec((_GB, 1), lambda i: (i, 0)),
            pl.BlockSpec((1, _OUT), lambda i: (0, 0)),
            pl.BlockSpec((_HID, _OUT), lambda i: (0, 0)),
        ],
        out_specs=pl.BlockSpec((_GB, _OUT), lambda i: (i, 0)),
        out_shape=jax.ShapeDtypeStruct((_N, _OUT), jnp.float32),
    )(g, hp, deg0, deg1, b2, W2)


def kernel(x, edge_index, W1, b1, W2, b2):
    degp = _deg_kernel(edge_index).reshape(_NC, _ND)
    deg0 = degp[0, :_N, None]
    deg1 = degp[1, :_N, None]

    h1p = _mm1(x, W1, deg0, deg1)
    agg1 = _agg64(h1p, edge_index)
    h2p = _mid(agg1, h1p, deg0, deg1,
               b1.reshape(1, _HID))
    agg2 = _agg64(h2p, edge_index)
    return _fin(agg2, h2p, deg0, deg1,
                b2.reshape(1, _OUT), W2)
